# baseline (speedup 1.0000x reference)
import functools

import jax
import jax.numpy as jnp
from jax import lax
from jax.experimental import pallas as pl
from jax.experimental.pallas import tpu as pltpu
from jax.experimental.pallas import tpu_sc as plsc

_NNZ = 196608
_HALF = _NNZ // 2
_K = 128
_G = 16
_T = _HALF // (_K * _G)
_NV = 4096
_W = 16
_AP = 16


def _sc_spmm(xs, vals_h, rows_h, cols_h):
    mesh = plsc.VectorSubcoreMesh(core_axis_name="core", subcore_axis_name="subcore")

    @functools.partial(
        pl.kernel,
        out_type=jax.ShapeDtypeStruct((_NV, 2 * 16 * _W), jnp.float32),
        mesh=mesh,
        compiler_params=pltpu.CompilerParams(needs_layout_passes=False,
                                             use_tc_tiling_on_sc=False),
        scratch_types=[
            pltpu.VMEM((_G, _K), jnp.int32),
            pltpu.VMEM((_G, _K), jnp.int32),
            pltpu.VMEM((_G, _K), jnp.float32),
            pltpu.VMEM((_G, _K), jnp.int32),
            pltpu.VMEM((4, _K, _W), jnp.float32),
            pltpu.VMEM((_NV, _AP), jnp.float32),
            pltpu.SemaphoreType.DMA,
            pltpu.SemaphoreType.DMA,
            pltpu.SemaphoreType.DMA,
            pltpu.SemaphoreType.DMA,
        ],
    )
    def k(xs_hbm, vals_hbm, rows_hbm, cols_hbm, z_hbm,
          cols_v, rows_v, vals_v, idx_v, gbuf, acc, s0, s1, s2, s3):
        sems = (s0, s1, s2, s3)
        cid = lax.axis_index("core")
        sid = lax.axis_index("subcore")

        @pl.loop(0, _NV)
        def _(r):
            acc[r, pl.ds(0, _W)] = jnp.zeros((_W,), jnp.float32)

        soff = jnp.zeros((16,), jnp.int32) + sid

        @pl.loop(0, _T)
        def _(t):
            pltpu.sync_copy(cols_hbm.at[cid, t], cols_v)
            pltpu.sync_copy(rows_hbm.at[cid, t], rows_v)
            pltpu.sync_copy(vals_hbm.at[cid, t], vals_v)

            @pl.loop(0, _G)
            def _(s):
                for h in range(_K // 16):
                    sl = pl.ds(h * 16, 16)
                    idx_v[s, sl] = cols_v[s, sl] * 16 + soff

            descs = [None] * _G
            for s in range(3):
                descs[s] = pltpu.async_copy(
                    xs_hbm.at[idx_v.at[s]], gbuf.at[s % 4], sems[s % 4])
            for s in range(_G):
                descs[s].wait()
                if s + 3 < _G:
                    descs[s + 3] = pltpu.async_copy(
                        xs_hbm.at[idx_v.at[s + 3]],
                        gbuf.at[(s + 3) % 4], sems[(s + 3) % 4])
                slot = s % 4

                ssplat = jnp.full((16,), s, jnp.int32)

                @pl.loop(0, _K // 16)
                def _(b):
                    rv = rows_v[s, pl.ds(b * 16, 16)]
                    for j in range(16):
                        e = b * 16 + j
                        val = plsc.load_gather(
                            vals_v, [ssplat, jnp.full((16,), e, jnp.int32)])
                        plsc.addupdate(acc.at[rv[j], pl.ds(0, _W)],
                                       gbuf[slot, e, :] * val)

        coloff = (cid * 16 + sid) * _W

        @pl.loop(0, _NV // _K)
        def _(blk):
            @pl.loop(0, _K)
            def _(r):
                gbuf[0, r, :] = acc[blk * _K + r, pl.ds(0, _W)]
            pltpu.sync_copy(gbuf.at[0],
                            z_hbm.at[pl.ds(blk * _K, _K),
                                     pl.ds(coloff, _W)])

    return k(xs, vals_h, rows_h, cols_h)


def _tc_body(z_ref, bd_ref, b_ref, o_ref):
    zsum = z_ref[:, :256] + z_ref[:, 256:]
    o_ref[...] = (
        jnp.dot(zsum, bd_ref[...], preferred_element_type=jnp.float32)
        + b_ref[...]
    )


def _tc_matmul(zp, bd, bias768):
    bm = 512
    return pl.pallas_call(
        _tc_body,
        grid=(_NV // bm,),
        in_specs=[
            pl.BlockSpec((bm, 512), lambda i: (i, 0)),
            pl.BlockSpec((256, 768), lambda i: (0, 0)),
            pl.BlockSpec((1, 768), lambda i: (0, 0)),
        ],
        out_specs=pl.BlockSpec((bm, 768), lambda i: (i, 0)),
        out_shape=jax.ShapeDtypeStruct((_NV, 768), jnp.float32),
    )(zp, bd, bias768)


def kernel(x, weight, bias, cheb_vals, cheb_rows, cheb_cols):
    xs = x.reshape(_NNZ, _W)
    cols_h = cheb_cols.reshape(2, _T, _G, _K)
    rows_h = cheb_rows.reshape(2, _T, _G, _K)
    vals_h = cheb_vals.reshape(2, _T, _G, _K)

    zp = _sc_spmm(xs, vals_h, rows_h, cols_h)

    wm = weight.reshape(32, 96)
    bd = (jnp.eye(8, dtype=jnp.float32)[:, None, :, None]
          * wm[None, :, None, :]).reshape(256, 768)
    bias768 = jnp.tile(bias, 24).reshape(1, 768)

    out = _tc_matmul(zp, bd, bias768)
    return out.reshape(-1, 32)


# speedup vs baseline: 1.0266x; 1.0266x over previous
import functools

import jax
import jax.numpy as jnp
from jax import lax
from jax.experimental import pallas as pl
from jax.experimental.pallas import tpu as pltpu
from jax.experimental.pallas import tpu_sc as plsc

_NNZ = 196608
_HALF = _NNZ // 2
_K = 128
_G = 16
_T = _HALF // (_K * _G)
_NV = 4096
_W = 16
_AP = 16


def _sc_spmm(xs, vals_h, rows_h, cols_h):
    mesh = plsc.VectorSubcoreMesh(core_axis_name="core", subcore_axis_name="subcore")

    @functools.partial(
        pl.kernel,
        out_type=jax.ShapeDtypeStruct((_NV, 2 * 16 * _W), jnp.float32),
        mesh=mesh,
        compiler_params=pltpu.CompilerParams(needs_layout_passes=False,
                                             use_tc_tiling_on_sc=False),
        scratch_types=[
            pltpu.VMEM((_G, _K), jnp.int32),
            pltpu.VMEM((_G, _K), jnp.int32),
            pltpu.VMEM((_G, _K), jnp.float32),
            pltpu.VMEM((_G, _K), jnp.int32),
            pltpu.VMEM((4, _K, _W), jnp.float32),
            pltpu.VMEM((_NV, _AP), jnp.float32),
            pltpu.SemaphoreType.DMA,
            pltpu.SemaphoreType.DMA,
            pltpu.SemaphoreType.DMA,
            pltpu.SemaphoreType.DMA,
        ],
    )
    def k(xs_hbm, vals_hbm, rows_hbm, cols_hbm, z_hbm,
          cols_v, rows_v, vals_v, idx_v, gbuf, acc, s0, s1, s2, s3):
        sems = (s0, s1, s2, s3)
        cid = lax.axis_index("core")
        sid = lax.axis_index("subcore")

        @pl.loop(0, _NV)
        def _(r):
            acc[r, pl.ds(0, _W)] = jnp.zeros((_W,), jnp.float32)

        soff = jnp.zeros((16,), jnp.int32) + sid

        @pl.loop(0, _T)
        def _(t):
            pltpu.sync_copy(cols_hbm.at[cid, t], cols_v)
            pltpu.sync_copy(rows_hbm.at[cid, t], rows_v)
            pltpu.sync_copy(vals_hbm.at[cid, t], vals_v)

            @pl.loop(0, _G)
            def _(s):
                for h in range(_K // 16):
                    sl = pl.ds(h * 16, 16)
                    idx_v[s, sl] = cols_v[s, sl] * 16 + soff

            descs = [None] * _G
            for s in range(3):
                descs[s] = pltpu.async_copy(
                    xs_hbm.at[idx_v.at[s]], gbuf.at[s % 4], sems[s % 4])
            for s in range(_G):
                descs[s].wait()
                if s + 3 < _G:
                    descs[s + 3] = pltpu.async_copy(
                        xs_hbm.at[idx_v.at[s + 3]],
                        gbuf.at[(s + 3) % 4], sems[(s + 3) % 4])
                slot = s % 4

                @pl.loop(0, _K // 16)
                def _(b):
                    rv = rows_v[s, pl.ds(b * 16, 16)]
                    vv = vals_v[s, pl.ds(b * 16, 16)]
                    for j in range(16):
                        e = b * 16 + j
                        plsc.addupdate(acc.at[rv[j], pl.ds(0, _W)],
                                       gbuf[slot, e, :] * vv[j])

        coloff = (cid * 16 + sid) * _W

        @pl.loop(0, _NV // _K)
        def _(blk):
            @pl.loop(0, _K)
            def _(r):
                gbuf[0, r, :] = acc[blk * _K + r, pl.ds(0, _W)]
            pltpu.sync_copy(gbuf.at[0],
                            z_hbm.at[pl.ds(blk * _K, _K),
                                     pl.ds(coloff, _W)])

    return k(xs, vals_h, rows_h, cols_h)


def _tc_body(z_ref, bd_ref, b_ref, o_ref):
    zsum = z_ref[:, :256] + z_ref[:, 256:]
    o_ref[...] = (
        jnp.dot(zsum, bd_ref[...], preferred_element_type=jnp.float32)
        + b_ref[...]
    )


def _tc_matmul(zp, bd, bias768):
    bm = 512
    return pl.pallas_call(
        _tc_body,
        grid=(_NV // bm,),
        in_specs=[
            pl.BlockSpec((bm, 512), lambda i: (i, 0)),
            pl.BlockSpec((256, 768), lambda i: (0, 0)),
            pl.BlockSpec((1, 768), lambda i: (0, 0)),
        ],
        out_specs=pl.BlockSpec((bm, 768), lambda i: (i, 0)),
        out_shape=jax.ShapeDtypeStruct((_NV, 768), jnp.float32),
    )(zp, bd, bias768)


def kernel(x, weight, bias, cheb_vals, cheb_rows, cheb_cols):
    xs = x.reshape(_NNZ, _W)
    cols_h = cheb_cols.reshape(2, _T, _G, _K)
    rows_h = cheb_rows.reshape(2, _T, _G, _K)
    vals_h = cheb_vals.reshape(2, _T, _G, _K)

    zp = _sc_spmm(xs, vals_h, rows_h, cols_h)

    wm = weight.reshape(32, 96)
    bd = (jnp.eye(8, dtype=jnp.float32)[:, None, :, None]
          * wm[None, :, None, :]).reshape(256, 768)
    bias768 = jnp.tile(bias, 24).reshape(1, 768)

    out = _tc_matmul(zp, bd, bias768)
    return out.reshape(-1, 32)
